# Initial kernel scaffold; baseline (speedup 1.0000x reference)
import jax
import jax.numpy as jnp
from jax.experimental import pallas as pl


def kernel(image_feat, text_feat, user_pref, edge_index, edge_weight, Wv, bv, Wt, bt, Wf):
    raise NotImplementedError("write your pallas kernel here")


# speedup vs baseline: 1.6266x; 1.6266x over previous
import functools

import jax
import jax.numpy as jnp
from jax.experimental import pallas as pl

NU = 25000
NI = 25000
N = NU + NI
DF = 512
DL = 64
N_LAYERS = 3

_BR = 1000


def _frontend_items_body(img_ref, txt_ref, Wv_ref, bv_ref, Wt_ref, bt_ref, out_ref):
    eps = 1e-12
    img = img_ref[...]
    n = jnp.sqrt(jnp.sum(img * img, axis=1, keepdims=True))
    img = img / jnp.maximum(n, eps)
    tv = jnp.dot(img, Wv_ref[...].T, preferred_element_type=jnp.float32) + bv_ref[...]
    nv = jnp.sqrt(jnp.sum(tv * tv, axis=1, keepdims=True))
    tv = tv / jnp.maximum(nv, eps)

    txt = txt_ref[...]
    n = jnp.sqrt(jnp.sum(txt * txt, axis=1, keepdims=True))
    txt = txt / jnp.maximum(n, eps)
    tt = jnp.dot(txt, Wt_ref[...].T, preferred_element_type=jnp.float32) + bt_ref[...]
    nt = jnp.sqrt(jnp.sum(tt * tt, axis=1, keepdims=True))
    tt = tt / jnp.maximum(nt, eps)

    out_ref[...] = tv + tt


def _frontend_users_body(up_ref, out_ref):
    eps = 1e-12
    up = up_ref[...]
    n = jnp.sqrt(jnp.sum(up * up, axis=1, keepdims=True))
    out_ref[...] = 2.0 * (up / jnp.maximum(n, eps))


def _combine_body(up_ref, u0, u1, u2, u3, i0, i1, i2, i3, Wf_ref, ue_ref, ie_ref):
    usum = u0[...] + u1[...] + u2[...] + u3[...]
    ue_ref[...] = up_ref[...] + usum * 0.125
    isum = (i0[...] + i1[...] + i2[...] + i3[...]) * 0.125
    ie_ref[...] = jnp.tanh(
        jnp.dot(isum, Wf_ref[...].T, preferred_element_type=jnp.float32))


def _spmm(edge_index, edge_weight, x):
    dst = edge_index[0]
    src = edge_index[1]
    return jax.ops.segment_sum(edge_weight[:, None] * x[src], dst, num_segments=N)


def kernel(image_feat, text_feat, user_pref, edge_index, edge_weight, Wv, bv, Wt, bt, Wf):
    grid = (NI // _BR,)
    row_spec = pl.BlockSpec((_BR, DF), lambda i: (i, 0))
    emb_spec = pl.BlockSpec((_BR, DL), lambda i: (i, 0))
    full_w = pl.BlockSpec((DL, DF), lambda i: (0, 0))
    full_b = pl.BlockSpec((DL,), lambda i: (0,))

    s0_items = pl.pallas_call(
        _frontend_items_body,
        grid=grid,
        in_specs=[row_spec, row_spec, full_w, full_b, full_w, full_b],
        out_specs=emb_spec,
        out_shape=jax.ShapeDtypeStruct((NI, DL), jnp.float32),
    )(image_feat, text_feat, Wv, bv, Wt, bt)

    s0_users = pl.pallas_call(
        _frontend_users_body,
        grid=(NU // _BR,),
        in_specs=[emb_spec],
        out_specs=emb_spec,
        out_shape=jax.ShapeDtypeStruct((NU, DL), jnp.float32),
    )(user_pref)

    e0 = jnp.concatenate([s0_users, s0_items], axis=0)
    e1 = _spmm(edge_index, edge_weight, e0)
    e2 = _spmm(edge_index, edge_weight, e1)
    e3 = _spmm(edge_index, edge_weight, e2)

    ff = pl.BlockSpec((DL, DL), lambda i: (0, 0))
    user_emb, item_emb = pl.pallas_call(
        _combine_body,
        grid=(NU // _BR,),
        in_specs=[emb_spec] * 9 + [ff],
        out_specs=[emb_spec, emb_spec],
        out_shape=[jax.ShapeDtypeStruct((NU, DL), jnp.float32),
                   jax.ShapeDtypeStruct((NI, DL), jnp.float32)],
    )(user_pref, e0[:NU], e1[:NU], e2[:NU], e3[:NU],
      e0[NU:], e1[NU:], e2[NU:], e3[NU:], Wf)

    return (user_emb, item_emb)


# speedup vs baseline: 5.3498x; 3.2890x over previous
import functools

import jax
import jax.numpy as jnp
from jax import lax
from jax.experimental import pallas as pl
from jax.experimental.pallas import tpu as pltpu
from jax.experimental.pallas import tpu_sc as plsc

NU = 25000
NI = 25000
N = NU + NI
E = 800000
DF = 512
DL = 64
DH = DL // 2

_BR = 1000

_EB = 125
_SUB = 4
_CHUNK = _EB * _SUB
_NCHUNK = E // 16 // _CHUNK
NP = 50048
_ROWS_PER_TILE = NP // 16


def _frontend_items_body(img_ref, txt_ref, Wv_ref, bv_ref, Wt_ref, bt_ref,
                         lo_ref, hi_ref):
    eps = 1e-12
    img = img_ref[...]
    n = jnp.sqrt(jnp.sum(img * img, axis=1, keepdims=True))
    img = img / jnp.maximum(n, eps)
    tv = jnp.dot(img, Wv_ref[...].T, preferred_element_type=jnp.float32) + bv_ref[...]
    nv = jnp.sqrt(jnp.sum(tv * tv, axis=1, keepdims=True))
    tv = tv / jnp.maximum(nv, eps)

    txt = txt_ref[...]
    n = jnp.sqrt(jnp.sum(txt * txt, axis=1, keepdims=True))
    txt = txt / jnp.maximum(n, eps)
    tt = jnp.dot(txt, Wt_ref[...].T, preferred_element_type=jnp.float32) + bt_ref[...]
    nt = jnp.sqrt(jnp.sum(tt * tt, axis=1, keepdims=True))
    tt = tt / jnp.maximum(nt, eps)

    s = tv + tt
    lo_ref[...] = s[:, :DH]
    hi_ref[...] = s[:, DH:]


def _frontend_users_body(up_ref, lo_ref, hi_ref):
    eps = 1e-12
    up = up_ref[...]
    n = jnp.sqrt(jnp.sum(up * up, axis=1, keepdims=True))
    s = 2.0 * (up / jnp.maximum(n, eps))
    lo_ref[...] = s[:, :DH]
    hi_ref[...] = s[:, DH:]


def _combine_body(up_ref, u0l, u0h, u1l, u1h, u2l, u2h, u3l, u3h,
                  i0l, i0h, i1l, i1h, i2l, i2h, i3l, i3h, Wf_ref,
                  ue_ref, ie_ref):
    ulo = u0l[...] + u1l[...] + u2l[...] + u3l[...]
    uhi = u0h[...] + u1h[...] + u2h[...] + u3h[...]
    usum = jnp.concatenate([ulo, uhi], axis=1)
    ue_ref[...] = up_ref[...] + usum * 0.125
    ilo = i0l[...] + i1l[...] + i2l[...] + i3l[...]
    ihi = i0h[...] + i1h[...] + i2h[...] + i3h[...]
    isum = jnp.concatenate([ilo, ihi], axis=1) * 0.125
    ie_ref[...] = jnp.tanh(
        jnp.dot(isum, Wf_ref[...].T, preferred_element_type=jnp.float32))


def _zero_rows(rows):
    z = jnp.zeros((16,), jnp.float32)

    def zb(r, car):
        rows[r, 0:16] = z
        rows[r, 16:32] = z
        return car

    lax.fori_loop(0, _CHUNK, zb, 0)


def _zero_acc_slice(s, rows, acc):
    base = s * _ROWS_PER_TILE
    off = 0
    left = _ROWS_PER_TILE
    while left > 0:
        cnt = min(_CHUNK, left)
        pltpu.sync_copy(rows.at[pl.ds(0, cnt), :], acc.at[pl.ds(base + off, cnt), :])
        off += cnt
        left -= cnt


def _copy_acc_out(s, acc, out_ref):
    base = s * _ROWS_PER_TILE
    off = 0
    left = _ROWS_PER_TILE
    while left > 0:
        cnt = min(_CHUNK, left)
        pltpu.sync_copy(acc.at[pl.ds(base + off, cnt), :],
                        out_ref.at[pl.ds(base + off, cnt), :])
        off += cnt
        left -= cnt


def _sc_spmm3_body(xlo, xhi, ei, w2, e1lo, e1hi, e2lo, e2hi, e3lo, e3hi,
                   dsti, srci, wbx, rows, acc, sem):
    c = lax.axis_index("c")
    s = lax.axis_index("s")

    _zero_rows(rows)
    _zero_acc_slice(s, rows, acc)
    plsc.subcore_barrier()

    def run_half(x0, o1, o2, o3):
        for src_ref, out_ref in ((x0, o1), (o1, o2), (o2, o3)):
            def chunk(k, car):
                roff = s * (_SUB * _NCHUNK) + k * _SUB
                eoff = roff * _EB
                pltpu.sync_copy(ei.at[0, pl.ds(roff, _SUB), :], dsti)
                pltpu.sync_copy(ei.at[1, pl.ds(roff, _SUB), :], srci)
                pltpu.sync_copy(w2.at[pl.ds(eoff, _CHUNK), :], wbx)
                copies = []
                for j in range(_SUB):
                    copies.append(pltpu.async_copy(
                        src_ref.at[srci.at[j]],
                        rows.at[pl.ds(j * _EB, _EB), :], sem))
                for cp in copies:
                    cp.wait()
                def sb(kk, car2):
                    wv = wbx[kk, 0:16]
                    rows[kk, 0:16] = rows[kk, 0:16] * wv
                    rows[kk, 16:32] = rows[kk, 16:32] * wv
                    return car2
                lax.fori_loop(0, _CHUNK, sb, 0)
                for j in range(_SUB):
                    pltpu.sync_copy(rows.at[pl.ds(j * _EB, _EB), :],
                                    acc.at[dsti.at[j]], add=True)
                return car

            lax.fori_loop(0, _NCHUNK, chunk, 0)
            plsc.subcore_barrier()
            _copy_acc_out(s, acc, out_ref)
            _zero_rows(rows)
            _zero_acc_slice(s, rows, acc)
            plsc.subcore_barrier()

    @pl.when(c == 0)
    def _():
        run_half(xlo, e1lo, e2lo, e3lo)

    @pl.when(c == 1)
    def _():
        run_half(xhi, e1hi, e2hi, e3hi)


def _sc_spmm3(xlo, xhi, ei, w2):
    f32 = jnp.float32
    out = jax.ShapeDtypeStruct((NP, DH), f32)
    fn = pl.kernel(
        _sc_spmm3_body,
        mesh=plsc.VectorSubcoreMesh(core_axis_name="c", subcore_axis_name="s"),
        out_type=[out] * 6,
        scratch_types=[
            pltpu.VMEM((_SUB, _EB), jnp.int32),
            pltpu.VMEM((_SUB, _EB), jnp.int32),
            pltpu.VMEM((_CHUNK, 16), f32),
            pltpu.VMEM((_CHUNK, DH), f32),
            pltpu.VMEM_SHARED((NP, DH), f32),
            pltpu.SemaphoreType.DMA,
        ],
        compiler_params=pltpu.CompilerParams(use_tc_tiling_on_sc=False),
    )
    return fn(xlo, xhi, ei, w2)


def kernel(image_feat, text_feat, user_pref, edge_index, edge_weight, Wv, bv, Wt, bt, Wf):
    f32 = jnp.float32
    row_spec = pl.BlockSpec((_BR, DF), lambda i: (i, 0))
    emb_spec = pl.BlockSpec((_BR, DL), lambda i: (i, 0))
    half_spec = pl.BlockSpec((_BR, DH), lambda i: (i, 0))
    full_w = pl.BlockSpec((DL, DF), lambda i: (0, 0))
    full_b = pl.BlockSpec((DL,), lambda i: (0,))

    i_lo, i_hi = pl.pallas_call(
        _frontend_items_body,
        grid=(NI // _BR,),
        in_specs=[row_spec, row_spec, full_w, full_b, full_w, full_b],
        out_specs=[half_spec, half_spec],
        out_shape=[jax.ShapeDtypeStruct((NI, DH), f32)] * 2,
    )(image_feat, text_feat, Wv, bv, Wt, bt)

    u_lo, u_hi = pl.pallas_call(
        _frontend_users_body,
        grid=(NU // _BR,),
        in_specs=[emb_spec],
        out_specs=[half_spec, half_spec],
        out_shape=[jax.ShapeDtypeStruct((NU, DH), f32)] * 2,
    )(user_pref)

    pad = jnp.zeros((NP - N, DH), f32)
    x_lo = jnp.concatenate([u_lo, i_lo, pad], axis=0)
    x_hi = jnp.concatenate([u_hi, i_hi, pad], axis=0)

    ei = edge_index.astype(jnp.int32).reshape(2, E // _EB, _EB)
    w2 = jnp.broadcast_to(edge_weight[:, None], (E, 16))

    e1lo, e1hi, e2lo, e2hi, e3lo, e3hi = _sc_spmm3(x_lo, x_hi, ei, w2)

    user_half = pl.BlockSpec((_BR, DH), lambda i: (i, 0))
    item_half = pl.BlockSpec((_BR, DH), lambda i: (i + NU // _BR, 0))
    ff = pl.BlockSpec((DL, DL), lambda i: (0, 0))
    user_emb, item_emb = pl.pallas_call(
        _combine_body,
        grid=(NU // _BR,),
        in_specs=[emb_spec,
                  user_half, user_half,
                  user_half, user_half, user_half, user_half,
                  user_half, user_half,
                  user_half, user_half,
                  item_half, item_half, item_half, item_half,
                  item_half, item_half,
                  ff],
        out_specs=[emb_spec, emb_spec],
        out_shape=[jax.ShapeDtypeStruct((NU, DL), f32),
                   jax.ShapeDtypeStruct((NI, DL), f32)],
    )(user_pref,
      u_lo, u_hi, e1lo, e1hi, e2lo, e2hi, e3lo, e3hi,
      i_lo, i_hi, e1lo, e1hi, e2lo, e2hi, e3lo, e3hi,
      Wf)

    return (user_emb, item_emb)


# speedup vs baseline: 5.7558x; 1.0759x over previous
import functools

import jax
import jax.numpy as jnp
from jax import lax
from jax.experimental import pallas as pl
from jax.experimental.pallas import tpu as pltpu
from jax.experimental.pallas import tpu_sc as plsc

NU = 25000
NI = 25000
N = NU + NI
E = 800000
DF = 512
DL = 64
DH = DL // 2

_BR = 1000

_EB = 125
_SUB = 4
_CHUNK = _EB * _SUB
_NCHUNK = E // 16 // _CHUNK
NP = 50048
_ROWS_PER_TILE = NP // 16


def _frontend_items_body(img_ref, txt_ref, Wv_ref, bv_ref, Wt_ref, bt_ref,
                         lo_ref, hi_ref):
    eps = 1e-12
    img = img_ref[...]
    n = jnp.sqrt(jnp.sum(img * img, axis=1, keepdims=True))
    img = img / jnp.maximum(n, eps)
    tv = jnp.dot(img, Wv_ref[...].T, preferred_element_type=jnp.float32) + bv_ref[...]
    nv = jnp.sqrt(jnp.sum(tv * tv, axis=1, keepdims=True))
    tv = tv / jnp.maximum(nv, eps)

    txt = txt_ref[...]
    n = jnp.sqrt(jnp.sum(txt * txt, axis=1, keepdims=True))
    txt = txt / jnp.maximum(n, eps)
    tt = jnp.dot(txt, Wt_ref[...].T, preferred_element_type=jnp.float32) + bt_ref[...]
    nt = jnp.sqrt(jnp.sum(tt * tt, axis=1, keepdims=True))
    tt = tt / jnp.maximum(nt, eps)

    s = tv + tt
    lo_ref[...] = s[:, :DH]
    hi_ref[...] = s[:, DH:]


def _frontend_users_body(up_ref, lo_ref, hi_ref):
    eps = 1e-12
    up = up_ref[...]
    n = jnp.sqrt(jnp.sum(up * up, axis=1, keepdims=True))
    s = 2.0 * (up / jnp.maximum(n, eps))
    lo_ref[...] = s[:, :DH]
    hi_ref[...] = s[:, DH:]


def _combine_body(up_ref, u0l, u0h, u1l, u1h, u2l, u2h, u3l, u3h,
                  i0l, i0h, i1l, i1h, i2l, i2h, i3l, i3h, Wf_ref,
                  ue_ref, ie_ref):
    ulo = u0l[...] + u1l[...] + u2l[...] + u3l[...]
    uhi = u0h[...] + u1h[...] + u2h[...] + u3h[...]
    usum = jnp.concatenate([ulo, uhi], axis=1)
    ue_ref[...] = up_ref[...] + usum * 0.125
    ilo = i0l[...] + i1l[...] + i2l[...] + i3l[...]
    ihi = i0h[...] + i1h[...] + i2h[...] + i3h[...]
    isum = jnp.concatenate([ilo, ihi], axis=1) * 0.125
    ie_ref[...] = jnp.tanh(
        jnp.dot(isum, Wf_ref[...].T, preferred_element_type=jnp.float32))


def _zero_rows(rows):
    z = jnp.zeros((16,), jnp.float32)

    def zb(r, car):
        rows[r, 0:16] = z
        rows[r, 16:32] = z
        return car

    lax.fori_loop(0, _CHUNK, zb, 0)


def _zero_acc_slice(s, rows, acc):
    base = s * _ROWS_PER_TILE
    off = 0
    left = _ROWS_PER_TILE
    while left > 0:
        cnt = min(_CHUNK, left)
        pltpu.sync_copy(rows.at[pl.ds(0, cnt), :], acc.at[pl.ds(base + off, cnt), :])
        off += cnt
        left -= cnt


def _copy_acc_out(s, acc, out_ref):
    base = s * _ROWS_PER_TILE
    off = 0
    left = _ROWS_PER_TILE
    while left > 0:
        cnt = min(_CHUNK, left)
        pltpu.sync_copy(acc.at[pl.ds(base + off, cnt), :],
                        out_ref.at[pl.ds(base + off, cnt), :])
        off += cnt
        left -= cnt


def _sc_spmm3_body(xlo, xhi, ei, w2, e1lo, e1hi, e2lo, e2hi, e3lo, e3hi,
                   dsti, srci, wbx, rows, acc, sem):
    c = lax.axis_index("c")
    s = lax.axis_index("s")

    _zero_rows(rows)
    _zero_acc_slice(s, rows, acc)
    plsc.subcore_barrier()

    def run_half(x0, o1, o2, o3):
        for src_ref, out_ref in ((x0, o1), (o1, o2), (o2, o3)):
            def chunk(k, car):
                roff = s * (_SUB * _NCHUNK) + k * _SUB
                eoff = roff * _EB
                pltpu.sync_copy(ei.at[0, pl.ds(roff, _SUB), :], dsti)
                pltpu.sync_copy(ei.at[1, pl.ds(roff, _SUB), :], srci)
                pltpu.sync_copy(w2.at[pl.ds(eoff, _CHUNK), :], wbx)
                copies = []
                for j in range(_SUB):
                    copies.append(pltpu.async_copy(
                        src_ref.at[srci.at[j]],
                        rows.at[pl.ds(j * _EB, _EB), :], sem.at[j]))
                for j in range(_SUB):
                    copies[j].wait()

                    def sb(kk, car2):
                        r = j * _EB + kk * 5
                        for i in range(5):
                            wv = wbx[r + i, 0:16]
                            rows[r + i, 0:16] = rows[r + i, 0:16] * wv
                            rows[r + i, 16:32] = rows[r + i, 16:32] * wv
                        return car2
                    lax.fori_loop(0, _EB // 5, sb, 0)
                    pltpu.sync_copy(rows.at[pl.ds(j * _EB, _EB), :],
                                    acc.at[dsti.at[j]], add=True)
                return car

            lax.fori_loop(0, _NCHUNK, chunk, 0)
            plsc.subcore_barrier()
            _copy_acc_out(s, acc, out_ref)
            _zero_rows(rows)
            _zero_acc_slice(s, rows, acc)
            plsc.subcore_barrier()

    @pl.when(c == 0)
    def _():
        run_half(xlo, e1lo, e2lo, e3lo)

    @pl.when(c == 1)
    def _():
        run_half(xhi, e1hi, e2hi, e3hi)


def _sc_spmm3(xlo, xhi, ei, w2):
    f32 = jnp.float32
    out = jax.ShapeDtypeStruct((NP, DH), f32)
    fn = pl.kernel(
        _sc_spmm3_body,
        mesh=plsc.VectorSubcoreMesh(core_axis_name="c", subcore_axis_name="s"),
        out_type=[out] * 6,
        scratch_types=[
            pltpu.VMEM((_SUB, _EB), jnp.int32),
            pltpu.VMEM((_SUB, _EB), jnp.int32),
            pltpu.VMEM((_CHUNK, 16), f32),
            pltpu.VMEM((_CHUNK, DH), f32),
            pltpu.VMEM_SHARED((NP, DH), f32),
            pltpu.SemaphoreType.DMA((_SUB,)),
        ],
        compiler_params=pltpu.CompilerParams(use_tc_tiling_on_sc=False),
    )
    return fn(xlo, xhi, ei, w2)


def kernel(image_feat, text_feat, user_pref, edge_index, edge_weight, Wv, bv, Wt, bt, Wf):
    f32 = jnp.float32
    row_spec = pl.BlockSpec((_BR, DF), lambda i: (i, 0))
    emb_spec = pl.BlockSpec((_BR, DL), lambda i: (i, 0))
    half_spec = pl.BlockSpec((_BR, DH), lambda i: (i, 0))
    full_w = pl.BlockSpec((DL, DF), lambda i: (0, 0))
    full_b = pl.BlockSpec((DL,), lambda i: (0,))

    i_lo, i_hi = pl.pallas_call(
        _frontend_items_body,
        grid=(NI // _BR,),
        in_specs=[row_spec, row_spec, full_w, full_b, full_w, full_b],
        out_specs=[half_spec, half_spec],
        out_shape=[jax.ShapeDtypeStruct((NI, DH), f32)] * 2,
    )(image_feat, text_feat, Wv, bv, Wt, bt)

    u_lo, u_hi = pl.pallas_call(
        _frontend_users_body,
        grid=(NU // _BR,),
        in_specs=[emb_spec],
        out_specs=[half_spec, half_spec],
        out_shape=[jax.ShapeDtypeStruct((NU, DH), f32)] * 2,
    )(user_pref)

    pad = jnp.zeros((NP - N, DH), f32)
    x_lo = jnp.concatenate([u_lo, i_lo, pad], axis=0)
    x_hi = jnp.concatenate([u_hi, i_hi, pad], axis=0)

    ei = edge_index.astype(jnp.int32).reshape(2, E // _EB, _EB)
    w2 = jnp.broadcast_to(edge_weight[:, None], (E, 16))

    e1lo, e1hi, e2lo, e2hi, e3lo, e3hi = _sc_spmm3(x_lo, x_hi, ei, w2)

    user_half = pl.BlockSpec((_BR, DH), lambda i: (i, 0))
    item_half = pl.BlockSpec((_BR, DH), lambda i: (i + NU // _BR, 0))
    ff = pl.BlockSpec((DL, DL), lambda i: (0, 0))
    user_emb, item_emb = pl.pallas_call(
        _combine_body,
        grid=(NU // _BR,),
        in_specs=[emb_spec,
                  user_half, user_half,
                  user_half, user_half, user_half, user_half,
                  user_half, user_half,
                  user_half, user_half,
                  item_half, item_half, item_half, item_half,
                  item_half, item_half,
                  ff],
        out_specs=[emb_spec, emb_spec],
        out_shape=[jax.ShapeDtypeStruct((NU, DL), f32),
                   jax.ShapeDtypeStruct((NI, DL), f32)],
    )(user_pref,
      u_lo, u_hi, e1lo, e1hi, e2lo, e2hi, e3lo, e3hi,
      i_lo, i_hi, e1lo, e1hi, e2lo, e2hi, e3lo, e3hi,
      Wf)

    return (user_emb, item_emb)


# speedup vs baseline: 8.8986x; 1.5460x over previous
import functools

import jax
import jax.numpy as jnp
from jax import lax
from jax.experimental import pallas as pl
from jax.experimental.pallas import tpu as pltpu
from jax.experimental.pallas import tpu_sc as plsc

NU = 25000
NI = 25000
N = NU + NI
E = 800000
DF = 512
DL = 64
DH = DL // 2

_BR = 1000

_EB = 80
_SUB = 5
_CHUNK = _EB * _SUB
_NCHUNK = E // 16 // _CHUNK
NP = 50048
_ROWS_PER_TILE = NP // 16


def _frontend_items_body(img_ref, txt_ref, Wv_ref, bv_ref, Wt_ref, bt_ref,
                         lo_ref, hi_ref):
    eps = 1e-12
    img = img_ref[...]
    n = jnp.sqrt(jnp.sum(img * img, axis=1, keepdims=True))
    img = img / jnp.maximum(n, eps)
    tv = jnp.dot(img, Wv_ref[...].T, preferred_element_type=jnp.float32) + bv_ref[...]
    nv = jnp.sqrt(jnp.sum(tv * tv, axis=1, keepdims=True))
    tv = tv / jnp.maximum(nv, eps)

    txt = txt_ref[...]
    n = jnp.sqrt(jnp.sum(txt * txt, axis=1, keepdims=True))
    txt = txt / jnp.maximum(n, eps)
    tt = jnp.dot(txt, Wt_ref[...].T, preferred_element_type=jnp.float32) + bt_ref[...]
    nt = jnp.sqrt(jnp.sum(tt * tt, axis=1, keepdims=True))
    tt = tt / jnp.maximum(nt, eps)

    s = tv + tt
    lo_ref[...] = s[:, :DH]
    hi_ref[...] = s[:, DH:]


def _frontend_users_body(up_ref, lo_ref, hi_ref):
    eps = 1e-12
    up = up_ref[...]
    n = jnp.sqrt(jnp.sum(up * up, axis=1, keepdims=True))
    s = 2.0 * (up / jnp.maximum(n, eps))
    lo_ref[...] = s[:, :DH]
    hi_ref[...] = s[:, DH:]


def _combine_body(up_ref, u0l, u0h, u1l, u1h, u2l, u2h, u3l, u3h,
                  i0l, i0h, i1l, i1h, i2l, i2h, i3l, i3h, Wf_ref,
                  ue_ref, ie_ref):
    ulo = u0l[...] + u1l[...] + u2l[...] + u3l[...]
    uhi = u0h[...] + u1h[...] + u2h[...] + u3h[...]
    usum = jnp.concatenate([ulo, uhi], axis=1)
    ue_ref[...] = up_ref[...] + usum * 0.125
    ilo = i0l[...] + i1l[...] + i2l[...] + i3l[...]
    ihi = i0h[...] + i1h[...] + i2h[...] + i3h[...]
    isum = jnp.concatenate([ilo, ihi], axis=1) * 0.125
    ie_ref[...] = jnp.tanh(
        jnp.dot(isum, Wf_ref[...].T, preferred_element_type=jnp.float32))


def _zero_rows(rows):
    z = jnp.zeros((16,), jnp.float32)

    def zb(r, car):
        rows[r, 0:16] = z
        rows[r, 16:32] = z
        return car

    lax.fori_loop(0, _CHUNK, zb, 0)


def _zero_acc_slice(s, rows, acc):
    base = s * _ROWS_PER_TILE
    off = 0
    left = _ROWS_PER_TILE
    while left > 0:
        cnt = min(_CHUNK, left)
        pltpu.sync_copy(rows.at[pl.ds(0, cnt), :], acc.at[pl.ds(base + off, cnt), :])
        off += cnt
        left -= cnt


def _copy_acc_out(s, acc, out_ref):
    base = s * _ROWS_PER_TILE
    off = 0
    left = _ROWS_PER_TILE
    while left > 0:
        cnt = min(_CHUNK, left)
        pltpu.sync_copy(acc.at[pl.ds(base + off, cnt), :],
                        out_ref.at[pl.ds(base + off, cnt), :])
        off += cnt
        left -= cnt


def _sc_spmm3_body(xlo, xhi, ei, w2, e1lo, e1hi, e2lo, e2hi, e3lo, e3hi,
                   dsti, srci, wbx, rows, acc, sem):
    c = lax.axis_index("c")
    s = lax.axis_index("s")

    _zero_rows(rows)
    _zero_acc_slice(s, rows, acc)
    plsc.subcore_barrier()

    def run_half(x0, o1, o2, o3):
        for src_ref, out_ref in ((x0, o1), (o1, o2), (o2, o3)):
            def chunk(k, car):
                roff = s * (_SUB * _NCHUNK) + k * _SUB
                pltpu.sync_copy(ei.at[0, pl.ds(roff, _SUB), :], dsti)
                pltpu.sync_copy(ei.at[1, pl.ds(roff, _SUB), :], srci)
                pltpu.sync_copy(w2.at[pl.ds(roff, _SUB), :], wbx)
                copies = []
                for j in range(_SUB):
                    copies.append(pltpu.async_copy(
                        src_ref.at[srci.at[j]],
                        rows.at[pl.ds(j * _EB, _EB), :], sem.at[j]))
                for j in range(_SUB):
                    copies[j].wait()

                    def sb(g, car2):
                        wv16 = wbx[j, pl.ds(g * 16, 16)]
                        base = j * _EB + g * 16
                        for i in range(16):
                            r = base + i
                            wv = wv16[i]
                            rows[r, 0:16] = rows[r, 0:16] * wv
                            rows[r, 16:32] = rows[r, 16:32] * wv
                        return car2
                    lax.fori_loop(0, _EB // 16, sb, 0)
                    pltpu.sync_copy(rows.at[pl.ds(j * _EB, _EB), :],
                                    acc.at[dsti.at[j]], add=True)
                return car

            lax.fori_loop(0, _NCHUNK, chunk, 0)
            plsc.subcore_barrier()
            _copy_acc_out(s, acc, out_ref)
            _zero_rows(rows)
            _zero_acc_slice(s, rows, acc)
            plsc.subcore_barrier()

    @pl.when(c == 0)
    def _():
        run_half(xlo, e1lo, e2lo, e3lo)

    @pl.when(c == 1)
    def _():
        run_half(xhi, e1hi, e2hi, e3hi)


def _sc_spmm3(xlo, xhi, ei, w2):
    f32 = jnp.float32
    out = jax.ShapeDtypeStruct((NP, DH), f32)
    fn = pl.kernel(
        _sc_spmm3_body,
        mesh=plsc.VectorSubcoreMesh(core_axis_name="c", subcore_axis_name="s"),
        out_type=[out] * 6,
        scratch_types=[
            pltpu.VMEM((_SUB, _EB), jnp.int32),
            pltpu.VMEM((_SUB, _EB), jnp.int32),
            pltpu.VMEM((_SUB, _EB), f32),
            pltpu.VMEM((_CHUNK, DH), f32),
            pltpu.VMEM_SHARED((NP, DH), f32),
            pltpu.SemaphoreType.DMA((_SUB,)),
        ],
        compiler_params=pltpu.CompilerParams(use_tc_tiling_on_sc=False),
    )
    return fn(xlo, xhi, ei, w2)


def kernel(image_feat, text_feat, user_pref, edge_index, edge_weight, Wv, bv, Wt, bt, Wf):
    f32 = jnp.float32
    row_spec = pl.BlockSpec((_BR, DF), lambda i: (i, 0))
    emb_spec = pl.BlockSpec((_BR, DL), lambda i: (i, 0))
    half_spec = pl.BlockSpec((_BR, DH), lambda i: (i, 0))
    full_w = pl.BlockSpec((DL, DF), lambda i: (0, 0))
    full_b = pl.BlockSpec((DL,), lambda i: (0,))

    i_lo, i_hi = pl.pallas_call(
        _frontend_items_body,
        grid=(NI // _BR,),
        in_specs=[row_spec, row_spec, full_w, full_b, full_w, full_b],
        out_specs=[half_spec, half_spec],
        out_shape=[jax.ShapeDtypeStruct((NI, DH), f32)] * 2,
    )(image_feat, text_feat, Wv, bv, Wt, bt)

    u_lo, u_hi = pl.pallas_call(
        _frontend_users_body,
        grid=(NU // _BR,),
        in_specs=[emb_spec],
        out_specs=[half_spec, half_spec],
        out_shape=[jax.ShapeDtypeStruct((NU, DH), f32)] * 2,
    )(user_pref)

    pad = jnp.zeros((NP - N, DH), f32)
    x_lo = jnp.concatenate([u_lo, i_lo, pad], axis=0)
    x_hi = jnp.concatenate([u_hi, i_hi, pad], axis=0)

    ei = edge_index.astype(jnp.int32).reshape(2, E // _EB, _EB)
    w2 = edge_weight.reshape(E // _EB, _EB)

    e1lo, e1hi, e2lo, e2hi, e3lo, e3hi = _sc_spmm3(x_lo, x_hi, ei, w2)

    user_half = pl.BlockSpec((_BR, DH), lambda i: (i, 0))
    item_half = pl.BlockSpec((_BR, DH), lambda i: (i + NU // _BR, 0))
    ff = pl.BlockSpec((DL, DL), lambda i: (0, 0))
    user_emb, item_emb = pl.pallas_call(
        _combine_body,
        grid=(NU // _BR,),
        in_specs=[emb_spec,
                  user_half, user_half,
                  user_half, user_half, user_half, user_half,
                  user_half, user_half,
                  user_half, user_half,
                  item_half, item_half, item_half, item_half,
                  item_half, item_half,
                  ff],
        out_specs=[emb_spec, emb_spec],
        out_shape=[jax.ShapeDtypeStruct((NU, DL), f32),
                   jax.ShapeDtypeStruct((NI, DL), f32)],
    )(user_pref,
      u_lo, u_hi, e1lo, e1hi, e2lo, e2hi, e3lo, e3hi,
      i_lo, i_hi, e1lo, e1hi, e2lo, e2hi, e3lo, e3hi,
      Wf)

    return (user_emb, item_emb)


# speedup vs baseline: 10.9621x; 1.2319x over previous
import functools

import jax
import jax.numpy as jnp
from jax import lax
from jax.experimental import pallas as pl
from jax.experimental.pallas import tpu as pltpu
from jax.experimental.pallas import tpu_sc as plsc

NU = 25000
NI = 25000
N = NU + NI
E = 800000
DF = 512
DL = 64
DH = DL // 2

_BR = 1000

_EB = 80
_SUB = 5
_CHUNK = _EB * _SUB
_NCHUNK = E // 16 // _CHUNK
NP = 50048
_ROWS_PER_TILE = NP // 16


def _frontend_items_body(img_ref, txt_ref, Wv_ref, bv_ref, Wt_ref, bt_ref,
                         lo_ref, hi_ref):
    eps = 1e-12
    img = img_ref[...]
    n = jnp.sqrt(jnp.sum(img * img, axis=1, keepdims=True))
    img = img / jnp.maximum(n, eps)
    tv = jnp.dot(img, Wv_ref[...].T, preferred_element_type=jnp.float32) + bv_ref[...]
    nv = jnp.sqrt(jnp.sum(tv * tv, axis=1, keepdims=True))
    tv = tv / jnp.maximum(nv, eps)

    txt = txt_ref[...]
    n = jnp.sqrt(jnp.sum(txt * txt, axis=1, keepdims=True))
    txt = txt / jnp.maximum(n, eps)
    tt = jnp.dot(txt, Wt_ref[...].T, preferred_element_type=jnp.float32) + bt_ref[...]
    nt = jnp.sqrt(jnp.sum(tt * tt, axis=1, keepdims=True))
    tt = tt / jnp.maximum(nt, eps)

    s = tv + tt
    lo_ref[...] = s[:, :DH]
    hi_ref[...] = s[:, DH:]


def _frontend_users_body(up_ref, lo_ref, hi_ref):
    eps = 1e-12
    up = up_ref[...]
    n = jnp.sqrt(jnp.sum(up * up, axis=1, keepdims=True))
    s = 2.0 * (up / jnp.maximum(n, eps))
    lo_ref[...] = s[:, :DH]
    hi_ref[...] = s[:, DH:]


def _combine_body(up_ref, u0l, u0h, u1l, u1h, u2l, u2h, u3l, u3h,
                  i0l, i0h, i1l, i1h, i2l, i2h, i3l, i3h, Wf_ref,
                  ue_ref, ie_ref):
    ulo = u0l[...] + u1l[...] + u2l[...] + u3l[...]
    uhi = u0h[...] + u1h[...] + u2h[...] + u3h[...]
    usum = jnp.concatenate([ulo, uhi], axis=1)
    ue_ref[...] = up_ref[...] + usum * 0.125
    ilo = i0l[...] + i1l[...] + i2l[...] + i3l[...]
    ihi = i0h[...] + i1h[...] + i2h[...] + i3h[...]
    isum = jnp.concatenate([ilo, ihi], axis=1) * 0.125
    ie_ref[...] = jnp.tanh(
        jnp.dot(isum, Wf_ref[...].T, preferred_element_type=jnp.float32))


def _zero_rows(rows):
    z = jnp.zeros((16,), jnp.float32)

    def zb(r, car):
        rows[r, 0:16] = z
        rows[r, 16:32] = z
        return car

    lax.fori_loop(0, _CHUNK, zb, 0)


def _zero_acc_slice(s, rows, acc):
    base = s * _ROWS_PER_TILE
    off = 0
    left = _ROWS_PER_TILE
    while left > 0:
        cnt = min(_CHUNK, left)
        pltpu.sync_copy(rows.at[pl.ds(0, cnt), :], acc.at[pl.ds(base + off, cnt), :])
        off += cnt
        left -= cnt


def _copy_acc_out(s, acc, out_ref):
    base = s * _ROWS_PER_TILE
    off = 0
    left = _ROWS_PER_TILE
    while left > 0:
        cnt = min(_CHUNK, left)
        pltpu.sync_copy(acc.at[pl.ds(base + off, cnt), :],
                        out_ref.at[pl.ds(base + off, cnt), :])
        off += cnt
        left -= cnt


def _sc_spmm3_body(xlo, xhi, ei, w2, e1lo, e1hi, e2lo, e2hi, e3lo, e3hi,
                   dsti, srci, wbx, rows, acc, sem):
    c = lax.axis_index("c")
    s = lax.axis_index("s")

    _zero_rows(rows)
    _zero_acc_slice(s, rows, acc)
    plsc.subcore_barrier()

    def run_half(x0, o1, o2, o3):
        for src_ref, out_ref in ((x0, o1), (o1, o2), (o2, o3)):
            def chunk(k, car):
                roff = s * (_SUB * _NCHUNK) + k * _SUB
                cp_d = pltpu.async_copy(ei.at[0, pl.ds(roff, _SUB), :], dsti,
                                        sem.at[_SUB])
                cp_s = pltpu.async_copy(ei.at[1, pl.ds(roff, _SUB), :], srci,
                                        sem.at[_SUB + 1])
                cp_w = pltpu.async_copy(w2.at[pl.ds(roff, _SUB), :], wbx,
                                        sem.at[_SUB + 2])
                cp_s.wait()
                copies = []
                for j in range(_SUB):
                    copies.append(pltpu.async_copy(
                        src_ref.at[srci.at[j]],
                        rows.at[pl.ds(j * _EB, _EB), :], sem.at[j]))
                cp_w.wait()
                cp_d.wait()
                for j in range(_SUB):
                    copies[j].wait()

                    def sb(g, car2):
                        wv16 = wbx[j, pl.ds(g * 16, 16)]
                        base = j * _EB + g * 16
                        for i in range(16):
                            r = base + i
                            wv = wv16[i]
                            rows[r, 0:16] = rows[r, 0:16] * wv
                            rows[r, 16:32] = rows[r, 16:32] * wv
                        return car2
                    lax.fori_loop(0, _EB // 16, sb, 0)
                    pltpu.sync_copy(rows.at[pl.ds(j * _EB, _EB), :],
                                    acc.at[dsti.at[j]], add=True)
                return car

            lax.fori_loop(0, _NCHUNK, chunk, 0)
            plsc.subcore_barrier()
            _copy_acc_out(s, acc, out_ref)
            _zero_rows(rows)
            _zero_acc_slice(s, rows, acc)
            plsc.subcore_barrier()

    @pl.when(c == 0)
    def _():
        run_half(xlo, e1lo, e2lo, e3lo)

    @pl.when(c == 1)
    def _():
        run_half(xhi, e1hi, e2hi, e3hi)


def _sc_spmm3(xlo, xhi, ei, w2):
    f32 = jnp.float32
    out = jax.ShapeDtypeStruct((NP, DH), f32)
    fn = pl.kernel(
        _sc_spmm3_body,
        mesh=plsc.VectorSubcoreMesh(core_axis_name="c", subcore_axis_name="s"),
        out_type=[out] * 6,
        scratch_types=[
            pltpu.VMEM((_SUB, _EB), jnp.int32),
            pltpu.VMEM((_SUB, _EB), jnp.int32),
            pltpu.VMEM((_SUB, _EB), f32),
            pltpu.VMEM((_CHUNK, DH), f32),
            pltpu.VMEM_SHARED((NP, DH), f32),
            pltpu.SemaphoreType.DMA((_SUB + 3,)),
        ],
        compiler_params=pltpu.CompilerParams(use_tc_tiling_on_sc=False),
    )
    return fn(xlo, xhi, ei, w2)


def kernel(image_feat, text_feat, user_pref, edge_index, edge_weight, Wv, bv, Wt, bt, Wf):
    f32 = jnp.float32
    row_spec = pl.BlockSpec((_BR, DF), lambda i: (i, 0))
    emb_spec = pl.BlockSpec((_BR, DL), lambda i: (i, 0))
    half_spec = pl.BlockSpec((_BR, DH), lambda i: (i, 0))
    full_w = pl.BlockSpec((DL, DF), lambda i: (0, 0))
    full_b = pl.BlockSpec((DL,), lambda i: (0,))

    i_lo, i_hi = pl.pallas_call(
        _frontend_items_body,
        grid=(NI // _BR,),
        in_specs=[row_spec, row_spec, full_w, full_b, full_w, full_b],
        out_specs=[half_spec, half_spec],
        out_shape=[jax.ShapeDtypeStruct((NI, DH), f32)] * 2,
    )(image_feat, text_feat, Wv, bv, Wt, bt)

    u_lo, u_hi = pl.pallas_call(
        _frontend_users_body,
        grid=(NU // _BR,),
        in_specs=[emb_spec],
        out_specs=[half_spec, half_spec],
        out_shape=[jax.ShapeDtypeStruct((NU, DH), f32)] * 2,
    )(user_pref)

    pad = jnp.zeros((NP - N, DH), f32)
    x_lo = jnp.concatenate([u_lo, i_lo, pad], axis=0)
    x_hi = jnp.concatenate([u_hi, i_hi, pad], axis=0)

    ei = edge_index.astype(jnp.int32).reshape(2, E // _EB, _EB)
    w2 = edge_weight.reshape(E // _EB, _EB)

    e1lo, e1hi, e2lo, e2hi, e3lo, e3hi = _sc_spmm3(x_lo, x_hi, ei, w2)

    user_half = pl.BlockSpec((_BR, DH), lambda i: (i, 0))
    item_half = pl.BlockSpec((_BR, DH), lambda i: (i + NU // _BR, 0))
    ff = pl.BlockSpec((DL, DL), lambda i: (0, 0))
    user_emb, item_emb = pl.pallas_call(
        _combine_body,
        grid=(NU // _BR,),
        in_specs=[emb_spec,
                  user_half, user_half,
                  user_half, user_half, user_half, user_half,
                  user_half, user_half,
                  user_half, user_half,
                  item_half, item_half, item_half, item_half,
                  item_half, item_half,
                  ff],
        out_specs=[emb_spec, emb_spec],
        out_shape=[jax.ShapeDtypeStruct((NU, DL), f32),
                   jax.ShapeDtypeStruct((NI, DL), f32)],
    )(user_pref,
      u_lo, u_hi, e1lo, e1hi, e2lo, e2hi, e3lo, e3hi,
      i_lo, i_hi, e1lo, e1hi, e2lo, e2hi, e3lo, e3hi,
      Wf)

    return (user_emb, item_emb)
